# baseline (speedup 1.0000x reference)
import functools

import jax
import jax.numpy as jnp
from jax import lax
from jax.experimental import pallas as pl
from jax.experimental.pallas import tpu as pltpu
from jax.experimental.pallas import tpu_sc as plsc

_NC, _NS, _L = 2, 16, 16
_NW = _NC * _NS
_B = 16384
_BPW = _B // _NW
_CHUNK = 128
_NCHUNK = _BPW // _CHUNK
_D = 32

_V = 1000000
_R = 999936
_TAIL = _V - _R
_BC = 512
_NBLK = _R // _BC
_BLK_PER_W = -(-_NBLK // _NW)


def _conv_body(tab_hbm, out_hbm, slab_v, rows_v, sem_in, sem_out):
    w = lax.axis_index("s") * _NC + lax.axis_index("c")
    iota = lax.iota(jnp.int32, _L)
    skew_consts = [iota * _D + ((d + iota) % _D) for d in range(_D)]

    def fire_in(b, buf):
        c0 = b * _BC
        for g in range(4):
            pltpu.async_copy(
                tab_hbm.at[pl.ds(g * 8, 8), pl.ds(c0, _BC)],
                slab_v.at[buf * 4 + g], sem_in)

    def wait_in(buf):
        for g in range(4):
            pltpu.make_async_copy(
                tab_hbm.at[pl.ds(0, 8), pl.ds(0, _BC)],
                slab_v.at[buf * 4 + g], sem_in).wait()

    def wait_out(buf):
        pltpu.make_async_copy(
            rows_v.at[pl.ds(buf * _BC * _D, _BC * _D)], out_hbm.at[pl.ds(0, _BC * _D)], sem_out).wait()

    fire_in(w, 0)

    def do_one(b, buf, first):
        @pl.when(b < _NBLK)
        def _():
            wait_in(buf)

            @pl.when(b + _NW < _NBLK)
            def _():
                fire_in(b + _NW, 1 - buf)

            rows_ref = rows_v.at[pl.ds(buf * _BC * _D, _BC * _D)]

            @plsc.parallel_loop(0, _BC // (2 * _L), unroll=2)
            def inner(c32):
                base = c32 * (2 * _L * _D)
                for d in range(_D):
                    v = slab_v[buf * 4 + d // 8, d % 8, pl.ds(c32 * 2 * _L, _L)]
                    plsc.store_scatter(rows_ref, [base + skew_consts[d]], v)
                for d in range(_D):
                    v = slab_v[buf * 4 + d // 8, d % 8,
                               pl.ds(c32 * 2 * _L + _L, _L)]
                    plsc.store_scatter(
                        rows_ref,
                        [base + _L * _D + skew_consts[(d + _L) % _D]], v)

            if first is None:
                wait_out(buf)
            else:
                @pl.when(jnp.logical_not(first))
                def _():
                    wait_out(buf)

            pltpu.async_copy(
                rows_v.at[pl.ds(buf * _BC * _D, _BC * _D)], out_hbm.at[pl.ds(b * _BC * _D, _BC * _D)],
                sem_out)

    def do_pair(k2, carry):
        b0 = (2 * k2) * _NW + w
        do_one(b0, 0, k2 == 0)
        do_one(b0 + _NW, 1, None)
        return carry

    lax.fori_loop(0, -(-_BLK_PER_W // 2), do_pair, 0)
    wait_out(0)


def _mf_body(uid_hbm, iid_hbm, utab_hbm, itab_hbm, utail_hbm, itail_hbm,
             out_hbm, uidx_v, iidx_v, utidx_v, itidx_v,
             uorig_v, iorig_v, urows_v, irows_v, out_v, sem):
    wid = lax.axis_index("s") * _NC + lax.axis_index("c")
    base = wid * _BPW

    for j in range(_NCHUNK):
        pltpu.sync_copy(uid_hbm.at[pl.ds(base + j * _CHUNK, _CHUNK)], uidx_v.at[j])
        pltpu.sync_copy(iid_hbm.at[pl.ds(base + j * _CHUNK, _CHUNK)], iidx_v.at[j])
        pltpu.sync_copy(uid_hbm.at[pl.ds(base + j * _CHUNK, _CHUNK)],
                        uorig_v.at[pl.ds(j * _CHUNK, _CHUNK)])
        pltpu.sync_copy(iid_hbm.at[pl.ds(base + j * _CHUNK, _CHUNK)],
                        iorig_v.at[pl.ds(j * _CHUNK, _CHUNK)])

    def split(j, carry):
        for k in range(_CHUNK // _L):
            sl = pl.ds(k * _L, _L)
            for idx_v, tidx_v in ((uidx_v, utidx_v), (iidx_v, itidx_v)):
                v = idx_v[j, sl]
                big = v >= _R
                tidx_v[j, sl] = jnp.where(big, v - _R, jnp.int32(-1))
                idx_v[j, sl] = jnp.where(big, jnp.int32(-1), v)
        return carry

    lax.fori_loop(0, _NCHUNK, split, 0)

    copies = []
    for j in range(_NCHUNK):
        dst = pl.ds(j * _CHUNK, _CHUNK)
        copies.append(pltpu.async_copy(
            utab_hbm.at[plsc.Indices(uidx_v.at[j], ignored_value=-1)],
            urows_v.at[dst], sem))
        copies.append(pltpu.async_copy(
            itab_hbm.at[plsc.Indices(iidx_v.at[j], ignored_value=-1)],
            irows_v.at[dst], sem))
        copies.append(pltpu.async_copy(
            utail_hbm.at[plsc.Indices(utidx_v.at[j], ignored_value=-1)],
            urows_v.at[dst], sem))
        copies.append(pltpu.async_copy(
            itail_hbm.at[plsc.Indices(itidx_v.at[j], ignored_value=-1)],
            irows_v.at[dst], sem))
    for c in copies:
        c.wait()

    lane = lax.iota(jnp.int32, _L)

    def block(t, carry):
        b0 = t * _L
        row_idx = b0 + lane
        uvec = uorig_v[pl.ds(b0, _L)]
        ivec = iorig_v[pl.ds(b0, _L)]
        acc = jnp.zeros((_L,), jnp.float32)
        for d in range(_D):
            ucc = (uvec + d) & (_D - 1)
            icc = (ivec + d) & (_D - 1)
            u = plsc.load_gather(urows_v, [row_idx, ucc])
            v = plsc.load_gather(irows_v, [row_idx, icc])
            acc = acc + u * v
        out_v[pl.ds(b0, _L)] = acc
        return carry

    lax.fori_loop(0, _BPW // _L, block, 0)

    pltpu.sync_copy(out_v, out_hbm.at[pl.ds(base, _BPW)])


@jax.jit
def kernel(user_ids, item_ids, user_table, item_table):
    mesh = plsc.VectorSubcoreMesh(core_axis_name="c", subcore_axis_name="s")

    conv = pl.kernel(
        _conv_body,
        out_type=jax.ShapeDtypeStruct((_R * _D,), jnp.float32),
        mesh=mesh,
        scratch_types=[
            pltpu.VMEM((8, 8, _BC), jnp.float32),
            pltpu.VMEM((2 * _BC * _D,), jnp.float32),
            pltpu.SemaphoreType.DMA,
            pltpu.SemaphoreType.DMA,
        ],
        compiler_params=pltpu.CompilerParams(needs_layout_passes=False),
    )
    u_lin = conv(user_table.T)
    i_lin = conv(item_table.T)

    run = pl.kernel(
        _mf_body,
        out_type=jax.ShapeDtypeStruct((_B,), jnp.float32),
        mesh=mesh,
        scratch_types=[
            pltpu.VMEM((_NCHUNK, _CHUNK), jnp.int32),
            pltpu.VMEM((_NCHUNK, _CHUNK), jnp.int32),
            pltpu.VMEM((_NCHUNK, _CHUNK), jnp.int32),
            pltpu.VMEM((_NCHUNK, _CHUNK), jnp.int32),
            pltpu.VMEM((_BPW,), jnp.int32),
            pltpu.VMEM((_BPW,), jnp.int32),
            pltpu.VMEM((_BPW, _D), jnp.float32),
            pltpu.VMEM((_BPW, _D), jnp.float32),
            pltpu.VMEM((_BPW,), jnp.float32),
            pltpu.SemaphoreType.DMA,
        ],
        compiler_params=pltpu.CompilerParams(
            needs_layout_passes=False, use_tc_tiling_on_sc=False),
    )
    dd = jnp.arange(_D)
    perm = (dd[None, :, None] + jnp.arange(_TAIL)[:, None, None]) % _D
    onehot = (perm == dd[None, None, :]).astype(jnp.float32)
    utail = jnp.einsum("jd,jdc->jc", user_table[_R:], onehot,
                       precision=lax.Precision.HIGHEST)
    itail = jnp.einsum("jd,jdc->jc", item_table[_R:], onehot,
                       precision=lax.Precision.HIGHEST)
    return run(user_ids, item_ids,
               u_lin.reshape(_R, _D), i_lin.reshape(_R, _D), utail, itail)


# speedup vs baseline: 1.2616x; 1.2616x over previous
import functools

import jax
import jax.numpy as jnp
from jax import lax
from jax.experimental import pallas as pl
from jax.experimental.pallas import tpu as pltpu
from jax.experimental.pallas import tpu_sc as plsc

_NC, _NS, _L = 2, 16, 16
_NW = _NC * _NS
_B = 16384
_BPW = _B // _NW
_CHUNK = 128
_NCHUNK = _BPW // _CHUNK
_D = 32

_V = 1000000
_R = 999936
_TAIL = _V - _R
_BC = 768
_NBLK = _R // _BC
_BLK_PER_W = -(-_NBLK // _NW)


def _conv_body(utab_hbm, itab_hbm, uout_hbm, iout_hbm, slab_v, rows_v,
               sem_u, sem_i, sem_ou, sem_oi):
    w = lax.axis_index("s") * _NC + lax.axis_index("c")
    iota = lax.iota(jnp.int32, _L)
    skew_consts = [iota * _D + ((d + iota) % _D) for d in range(_D)]

    def fire_in(tab_hbm, b, buf, sem):
        c0 = b * _BC
        for g in range(4):
            pltpu.async_copy(
                tab_hbm.at[pl.ds(g * 8, 8), pl.ds(c0, _BC)],
                slab_v.at[buf * 4 + g], sem)

    def wait_in(tab_hbm, buf, sem):
        for g in range(4):
            pltpu.make_async_copy(
                tab_hbm.at[pl.ds(0, 8), pl.ds(0, _BC)],
                slab_v.at[buf * 4 + g], sem).wait()

    def wait_out(out_hbm, buf, sem):
        pltpu.make_async_copy(
            rows_v.at[pl.ds(buf * _BC * _D, _BC * _D)],
            out_hbm.at[pl.ds(0, _BC * _D)], sem).wait()

    fire_in(utab_hbm, w, 0, sem_u)
    fire_in(itab_hbm, w, 1, sem_i)

    def do_one(tab_hbm, out_hbm, b, buf, k, sem, sem_o):
        @pl.when(b < _NBLK)
        def _():
            wait_in(tab_hbm, buf, sem)
            rows_ref = rows_v.at[pl.ds(buf * _BC * _D, _BC * _D)]

            @plsc.parallel_loop(0, _BC // (2 * _L), unroll=2)
            def inner(c32):
                base = c32 * (2 * _L * _D)
                for d in range(_D):
                    v = slab_v[buf * 4 + d // 8, d % 8, pl.ds(c32 * 2 * _L, _L)]
                    plsc.store_scatter(rows_ref, [base + skew_consts[d]], v)
                for d in range(_D):
                    v = slab_v[buf * 4 + d // 8, d % 8,
                               pl.ds(c32 * 2 * _L + _L, _L)]
                    plsc.store_scatter(
                        rows_ref,
                        [base + _L * _D + skew_consts[(d + _L) % _D]], v)

            @pl.when(k > 0)
            def _():
                wait_out(out_hbm, buf, sem_o)

            pltpu.async_copy(
                rows_ref, out_hbm.at[pl.ds(b * _BC * _D, _BC * _D)], sem_o)

            @pl.when(b + _NW < _NBLK)
            def _():
                fire_in(tab_hbm, b + _NW, buf, sem)

    def do_block(k, carry):
        b = k * _NW + w
        do_one(utab_hbm, uout_hbm, b, 0, k, sem_u, sem_ou)
        do_one(itab_hbm, iout_hbm, b, 1, k, sem_i, sem_oi)
        return carry

    lax.fori_loop(0, _BLK_PER_W, do_block, 0)
    wait_out(uout_hbm, 0, sem_ou)
    wait_out(iout_hbm, 1, sem_oi)


def _mf_body(uid_hbm, iid_hbm, utab_hbm, itab_hbm, utail_hbm, itail_hbm,
             out_hbm, uidx_v, iidx_v, utidx_v, itidx_v,
             uorig_v, iorig_v, urows_v, irows_v, out_v, sem):
    wid = lax.axis_index("s") * _NC + lax.axis_index("c")
    base = wid * _BPW

    for j in range(_NCHUNK):
        pltpu.sync_copy(uid_hbm.at[pl.ds(base + j * _CHUNK, _CHUNK)], uidx_v.at[j])
        pltpu.sync_copy(iid_hbm.at[pl.ds(base + j * _CHUNK, _CHUNK)], iidx_v.at[j])
        pltpu.sync_copy(uid_hbm.at[pl.ds(base + j * _CHUNK, _CHUNK)],
                        uorig_v.at[pl.ds(j * _CHUNK, _CHUNK)])
        pltpu.sync_copy(iid_hbm.at[pl.ds(base + j * _CHUNK, _CHUNK)],
                        iorig_v.at[pl.ds(j * _CHUNK, _CHUNK)])

    def split(j, carry):
        for k in range(_CHUNK // _L):
            sl = pl.ds(k * _L, _L)
            for idx_v, tidx_v in ((uidx_v, utidx_v), (iidx_v, itidx_v)):
                v = idx_v[j, sl]
                big = v >= _R
                tidx_v[j, sl] = jnp.where(big, v - _R, jnp.int32(-1))
                idx_v[j, sl] = jnp.where(big, jnp.int32(-1), v)
        return carry

    lax.fori_loop(0, _NCHUNK, split, 0)

    copies = []
    for j in range(_NCHUNK):
        dst = pl.ds(j * _CHUNK, _CHUNK)
        copies.append(pltpu.async_copy(
            utab_hbm.at[plsc.Indices(uidx_v.at[j], ignored_value=-1)],
            urows_v.at[dst], sem))
        copies.append(pltpu.async_copy(
            itab_hbm.at[plsc.Indices(iidx_v.at[j], ignored_value=-1)],
            irows_v.at[dst], sem))
        copies.append(pltpu.async_copy(
            utail_hbm.at[plsc.Indices(utidx_v.at[j], ignored_value=-1)],
            urows_v.at[dst], sem))
        copies.append(pltpu.async_copy(
            itail_hbm.at[plsc.Indices(itidx_v.at[j], ignored_value=-1)],
            irows_v.at[dst], sem))
    for c in copies:
        c.wait()

    lane = lax.iota(jnp.int32, _L)

    def block(t, carry):
        b0 = t * _L
        row_idx = b0 + lane
        uvec = uorig_v[pl.ds(b0, _L)]
        ivec = iorig_v[pl.ds(b0, _L)]
        acc = jnp.zeros((_L,), jnp.float32)
        for d in range(_D):
            ucc = (uvec + d) & (_D - 1)
            icc = (ivec + d) & (_D - 1)
            u = plsc.load_gather(urows_v, [row_idx, ucc])
            v = plsc.load_gather(irows_v, [row_idx, icc])
            acc = acc + u * v
        out_v[pl.ds(b0, _L)] = acc
        return carry

    lax.fori_loop(0, _BPW // _L, block, 0)

    pltpu.sync_copy(out_v, out_hbm.at[pl.ds(base, _BPW)])


@jax.jit
def kernel(user_ids, item_ids, user_table, item_table):
    mesh = plsc.VectorSubcoreMesh(core_axis_name="c", subcore_axis_name="s")

    conv = pl.kernel(
        _conv_body,
        out_type=(jax.ShapeDtypeStruct((_R * _D,), jnp.float32),
                  jax.ShapeDtypeStruct((_R * _D,), jnp.float32)),
        mesh=mesh,
        scratch_types=[
            pltpu.VMEM((8, 8, _BC), jnp.float32),
            pltpu.VMEM((2 * _BC * _D,), jnp.float32),
            pltpu.SemaphoreType.DMA,
            pltpu.SemaphoreType.DMA,
            pltpu.SemaphoreType.DMA,
            pltpu.SemaphoreType.DMA,
        ],
        compiler_params=pltpu.CompilerParams(needs_layout_passes=False),
    )
    u_lin, i_lin = conv(user_table.T, item_table.T)

    run = pl.kernel(
        _mf_body,
        out_type=jax.ShapeDtypeStruct((_B,), jnp.float32),
        mesh=mesh,
        scratch_types=[
            pltpu.VMEM((_NCHUNK, _CHUNK), jnp.int32),
            pltpu.VMEM((_NCHUNK, _CHUNK), jnp.int32),
            pltpu.VMEM((_NCHUNK, _CHUNK), jnp.int32),
            pltpu.VMEM((_NCHUNK, _CHUNK), jnp.int32),
            pltpu.VMEM((_BPW,), jnp.int32),
            pltpu.VMEM((_BPW,), jnp.int32),
            pltpu.VMEM((_BPW, _D), jnp.float32),
            pltpu.VMEM((_BPW, _D), jnp.float32),
            pltpu.VMEM((_BPW,), jnp.float32),
            pltpu.SemaphoreType.DMA,
        ],
        compiler_params=pltpu.CompilerParams(
            needs_layout_passes=False, use_tc_tiling_on_sc=False),
    )
    dd = jnp.arange(_D)
    perm = (dd[None, :, None] + jnp.arange(_TAIL)[:, None, None]) % _D
    onehot = (perm == dd[None, None, :]).astype(jnp.float32)
    utail = jnp.einsum("jd,jdc->jc", user_table[_R:], onehot,
                       precision=lax.Precision.HIGHEST)
    itail = jnp.einsum("jd,jdc->jc", item_table[_R:], onehot,
                       precision=lax.Precision.HIGHEST)
    return run(user_ids, item_ids,
               u_lin.reshape(_R, _D), i_lin.reshape(_R, _D), utail, itail)
